# baseline (speedup 1.0000x reference)
import jax
import jax.numpy as jnp
from jax import lax
from jax.experimental import pallas as pl
from jax.experimental.pallas import tpu as pltpu
from jax.experimental.pallas import tpu_sc as plsc

VOCAB = 1000000
EMBED_DIM = 64
BATCH = 4096
HIST = 200

NUM_CORES = 2
NUM_SUBCORES = 16

NW = NUM_CORES * NUM_SUBCORES
ROWS_PER_W = BATCH // NW
R = 4
NSTEP = ROWS_PER_W // R
NPAIR = (NSTEP - 2) // 2


def _emb_body(x_hbm, table_hbm, out_hbm,
              idx0, idx1, rows0, rows1, gsem0, gsem1, ssem0, ssem1):
    wid = lax.axis_index("s") * NUM_CORES + lax.axis_index("c")
    base = wid * ROWS_PER_W

    idxs = (idx0, idx1)
    bufs = (rows0, rows1)
    gsems = (gsem0, gsem1)
    ssems = (ssem0, ssem1)

    def start_gather(s, b):
        pltpu.sync_copy(x_hbm.at[pl.ds(base + s * R, R)], idxs[b])
        for j in range(R):
            pltpu.async_copy(table_hbm.at[idxs[b].at[j]], bufs[b].at[j],
                             gsems[b])

    def wait_gather(b):
        pltpu.make_async_copy(
            out_hbm.at[pl.ds(base, R)], bufs[b], gsems[b]).wait()

    def start_store(s, b):
        pltpu.async_copy(bufs[b], out_hbm.at[pl.ds(base + s * R, R)],
                         ssems[b])

    def wait_store(b):
        pltpu.make_async_copy(
            bufs[b], out_hbm.at[pl.ds(base, R)], ssems[b]).wait()

    start_gather(0, 0)
    wait_gather(0)
    start_gather(1, 1)
    start_store(0, 0)

    @pl.loop(0, NPAIR)
    def _pair(p):
        s = 1 + 2 * p
        for b in (1, 0):
            wait_gather(b)
            wait_store(1 - b)
            start_gather(s + 1, 1 - b)
            start_store(s, b)
            s = s + 1

    b_last = (NSTEP - 1) % 2
    wait_gather(b_last)
    wait_store(1 - b_last)
    start_store(NSTEP - 1, b_last)
    wait_store(b_last)


@jax.jit
def _embedding_sc(x, table):
    mesh = plsc.VectorSubcoreMesh(
        core_axis_name="c", subcore_axis_name="s",
        num_cores=NUM_CORES, num_subcores=NUM_SUBCORES)
    return pl.kernel(
        _emb_body,
        out_type=jax.ShapeDtypeStruct((BATCH, HIST, EMBED_DIM), jnp.float32),
        mesh=mesh,
        scratch_types=[
            pltpu.VMEM((R, HIST), jnp.int32),
            pltpu.VMEM((R, HIST), jnp.int32),
            pltpu.VMEM((R, HIST, EMBED_DIM), jnp.float32),
            pltpu.VMEM((R, HIST, EMBED_DIM), jnp.float32),
            pltpu.SemaphoreType.DMA,
            pltpu.SemaphoreType.DMA,
            pltpu.SemaphoreType.DMA,
            pltpu.SemaphoreType.DMA,
        ],
        compiler_params=pltpu.CompilerParams(use_tc_tiling_on_sc=False),
    )(x, table)


def kernel(x, table):
    return _embedding_sc(x, table)


# speedup vs baseline: 1.0174x; 1.0174x over previous
import jax
import jax.numpy as jnp
from jax import lax
from jax.experimental import pallas as pl
from jax.experimental.pallas import tpu as pltpu
from jax.experimental.pallas import tpu_sc as plsc

VOCAB = 1000000
EMBED_DIM = 64
BATCH = 4096
HIST = 200

NUM_CORES = 2
NUM_SUBCORES = 16
NW = NUM_CORES * NUM_SUBCORES

TOTAL = BATCH * HIST
B_PER_W = TOTAL // NW
IDX_ROWS = B_PER_W // 128

G = 512
NGPER = G // 128
NSTEP = B_PER_W // G
NPAIR = (NSTEP - 2) // 2


SAMP_PER_W = BATCH // NW


def _depad_body(x_hbm, x2_hbm, buf, sem):
    w = lax.axis_index("s") * NUM_CORES + lax.axis_index("c")
    s0 = w * SAMP_PER_W
    pltpu.sync_copy(x_hbm.at[pl.ds(s0, SAMP_PER_W)], buf)
    for j in range(SAMP_PER_W):
        pltpu.async_copy(
            buf.at[j], x2_hbm.at[pl.ds((s0 + j) * HIST, HIST)], sem)
    for j in range(SAMP_PER_W):
        pltpu.make_async_copy(
            buf.at[0], x2_hbm.at[pl.ds(s0 * HIST, HIST)], sem).wait()


def _repack(x):
    mesh = plsc.VectorSubcoreMesh(
        core_axis_name="c", subcore_axis_name="s",
        num_cores=NUM_CORES, num_subcores=NUM_SUBCORES)
    return pl.kernel(
        _depad_body,
        out_type=jax.ShapeDtypeStruct((TOTAL,), jnp.int32),
        mesh=mesh,
        scratch_types=[
            pltpu.VMEM((SAMP_PER_W, HIST), jnp.int32),
            pltpu.SemaphoreType.DMA,
        ],
        compiler_params=pltpu.CompilerParams(use_tc_tiling_on_sc=True),
    )(x)


def _gather_body(x2_hbm, table_hbm, out_hbm,
                 idx_v, rows0, rows1, gsem0, gsem1, ssem0, ssem1):
    w = lax.axis_index("s") * NUM_CORES + lax.axis_index("c")
    base = w * B_PER_W

    pltpu.sync_copy(x2_hbm.at[pl.ds(base, B_PER_W)], idx_v)

    bufs = (rows0, rows1)
    gsems = (gsem0, gsem1)
    ssems = (ssem0, ssem1)

    def start_gather(s, b):
        for j in range(NGPER):
            pltpu.async_copy(
                table_hbm.at[idx_v.at[pl.ds((s * NGPER + j) * 128, 128)]],
                bufs[b].at[pl.ds(j * 128, 128)],
                gsems[b])

    def wait_gather(b):
        pltpu.make_async_copy(
            out_hbm.at[pl.ds(base, G)], bufs[b], gsems[b]).wait()

    def start_store(s, b):
        pltpu.async_copy(bufs[b], out_hbm.at[pl.ds(base + s * G, G)],
                         ssems[b])

    def wait_store(b):
        pltpu.make_async_copy(
            bufs[b], out_hbm.at[pl.ds(base, G)], ssems[b]).wait()

    start_gather(0, 0)
    wait_gather(0)
    start_gather(1, 1)
    start_store(0, 0)

    @pl.loop(0, NPAIR)
    def _pair(p):
        s = 1 + 2 * p
        for b in (1, 0):
            wait_gather(b)
            wait_store(1 - b)
            start_gather(s + 1, 1 - b)
            start_store(s, b)
            s = s + 1

    b_last = (NSTEP - 1) % 2
    wait_gather(b_last)
    wait_store(1 - b_last)
    start_store(NSTEP - 1, b_last)
    wait_store(b_last)


def _gather(x2, table):
    mesh = plsc.VectorSubcoreMesh(
        core_axis_name="c", subcore_axis_name="s",
        num_cores=NUM_CORES, num_subcores=NUM_SUBCORES)
    return pl.kernel(
        _gather_body,
        out_type=jax.ShapeDtypeStruct((TOTAL, EMBED_DIM), jnp.float32),
        mesh=mesh,
        scratch_types=[
            pltpu.VMEM((B_PER_W,), jnp.int32),
            pltpu.VMEM((G, EMBED_DIM), jnp.float32),
            pltpu.VMEM((G, EMBED_DIM), jnp.float32),
            pltpu.SemaphoreType.DMA,
            pltpu.SemaphoreType.DMA,
            pltpu.SemaphoreType.DMA,
            pltpu.SemaphoreType.DMA,
        ],
        compiler_params=pltpu.CompilerParams(use_tc_tiling_on_sc=False),
    )(x2, table)


@jax.jit
def _embedding_sc(x, table):
    out2 = _gather(x.reshape(TOTAL), table)
    out2 = jax.lax.optimization_barrier(out2)
    return out2.reshape(BATCH, HIST, EMBED_DIM)


def kernel(x, table):
    return _embedding_sc(x, table)


# speedup vs baseline: 1.0181x; 1.0008x over previous
import jax
import jax.numpy as jnp
from jax import lax
from jax.experimental import pallas as pl
from jax.experimental.pallas import tpu as pltpu
from jax.experimental.pallas import tpu_sc as plsc

VOCAB = 1000000
EMBED_DIM = 64
BATCH = 4096
HIST = 200

NUM_CORES = 2
NUM_SUBCORES = 16
NW = NUM_CORES * NUM_SUBCORES

TOTAL = BATCH * HIST
B_PER_W = TOTAL // NW
IDX_ROWS = B_PER_W // 128

G = 512
NGPER = G // 128
NSTEP = B_PER_W // G
NPAIR = (NSTEP - 2) // 2


SAMP_PER_W = BATCH // NW


_COLS = tuple(range(0, HIST - 16, 16)) + (HIST - 16,)


def _depad_body(x_hbm, x2_hbm, buf, flat, sem):
    w = lax.axis_index("s") * NUM_CORES + lax.axis_index("c")
    s0 = w * SAMP_PER_W
    pltpu.sync_copy(x_hbm.at[pl.ds(s0, SAMP_PER_W)], buf)

    @pl.loop(0, SAMP_PER_W)
    def _row(j):
        for c in _COLS:
            flat[pl.ds(j * HIST + c, 16)] = buf[j, pl.ds(c, 16)]

    pltpu.sync_copy(flat, x2_hbm.at[pl.ds(s0 * HIST, SAMP_PER_W * HIST)])


def _repack(x):
    mesh = plsc.VectorSubcoreMesh(
        core_axis_name="c", subcore_axis_name="s",
        num_cores=NUM_CORES, num_subcores=NUM_SUBCORES)
    return pl.kernel(
        _depad_body,
        out_type=jax.ShapeDtypeStruct((TOTAL,), jnp.int32),
        mesh=mesh,
        scratch_types=[
            pltpu.VMEM((SAMP_PER_W, HIST), jnp.int32),
            pltpu.VMEM((SAMP_PER_W * HIST,), jnp.int32),
            pltpu.SemaphoreType.DMA,
        ],
        compiler_params=pltpu.CompilerParams(use_tc_tiling_on_sc=True),
    )(x)


def _gather_body(x2_hbm, table_hbm, out_hbm,
                 idx_v, rows0, rows1, gsem0, gsem1, ssem0, ssem1):
    w = lax.axis_index("s") * NUM_CORES + lax.axis_index("c")
    base = w * B_PER_W

    pltpu.sync_copy(x2_hbm.at[pl.ds(base, B_PER_W)], idx_v)

    bufs = (rows0, rows1)
    gsems = (gsem0, gsem1)
    ssems = (ssem0, ssem1)

    def start_gather(s, b):
        for j in range(NGPER):
            pltpu.async_copy(
                table_hbm.at[idx_v.at[pl.ds((s * NGPER + j) * 128, 128)]],
                bufs[b].at[pl.ds(j * 128, 128)],
                gsems[b])

    def wait_gather(b):
        pltpu.make_async_copy(
            out_hbm.at[pl.ds(base, G)], bufs[b], gsems[b]).wait()

    def start_store(s, b):
        pltpu.async_copy(bufs[b], out_hbm.at[pl.ds(base + s * G, G)],
                         ssems[b])

    def wait_store(b):
        pltpu.make_async_copy(
            bufs[b], out_hbm.at[pl.ds(base, G)], ssems[b]).wait()

    start_gather(0, 0)
    wait_gather(0)
    start_gather(1, 1)
    start_store(0, 0)

    @pl.loop(0, NPAIR)
    def _pair(p):
        s = 1 + 2 * p
        for b in (1, 0):
            wait_gather(b)
            wait_store(1 - b)
            start_gather(s + 1, 1 - b)
            start_store(s, b)
            s = s + 1

    b_last = (NSTEP - 1) % 2
    wait_gather(b_last)
    wait_store(1 - b_last)
    start_store(NSTEP - 1, b_last)
    wait_store(b_last)


def _gather(x2, table):
    mesh = plsc.VectorSubcoreMesh(
        core_axis_name="c", subcore_axis_name="s",
        num_cores=NUM_CORES, num_subcores=NUM_SUBCORES)
    return pl.kernel(
        _gather_body,
        out_type=jax.ShapeDtypeStruct((TOTAL, EMBED_DIM), jnp.float32),
        mesh=mesh,
        scratch_types=[
            pltpu.VMEM((B_PER_W,), jnp.int32),
            pltpu.VMEM((G, EMBED_DIM), jnp.float32),
            pltpu.VMEM((G, EMBED_DIM), jnp.float32),
            pltpu.SemaphoreType.DMA,
            pltpu.SemaphoreType.DMA,
            pltpu.SemaphoreType.DMA,
            pltpu.SemaphoreType.DMA,
        ],
        compiler_params=pltpu.CompilerParams(use_tc_tiling_on_sc=False),
    )(x2, table)


@jax.jit
def _embedding_sc(x, table):
    out2 = _gather(_repack(x), table)
    out2 = jax.lax.optimization_barrier(out2)
    return out2.reshape(BATCH, HIST, EMBED_DIM)


def kernel(x, table):
    return _embedding_sc(x, table)


# speedup vs baseline: 1.2785x; 1.2557x over previous
import jax
import jax.numpy as jnp
from jax import lax
from jax.experimental import layout as jex_layout
from jax.experimental import pallas as pl
from jax.experimental.pallas import tpu as pltpu
from jax.experimental.pallas import tpu_sc as plsc

VOCAB = 1000000
EMBED_DIM = 64
BATCH = 4096
HIST = 200

NUM_CORES = 2
NUM_SUBCORES = 16
NW = NUM_CORES * NUM_SUBCORES

TOTAL = BATCH * HIST
B_PER_W = TOTAL // NW
IDX_ROWS = B_PER_W // 128

G = 512
NGPER = G // 128
NSTEP = B_PER_W // G
NPAIR = (NSTEP - 2) // 2


SAMP_PER_W = BATCH // NW


_COLS = tuple(range(0, HIST - 16, 16)) + (HIST - 16,)


def _depad_body(x_hbm, x2_hbm, buf, flat, sem):
    w = lax.axis_index("s") * NUM_CORES + lax.axis_index("c")
    s0 = w * SAMP_PER_W
    pltpu.sync_copy(x_hbm.at[pl.ds(s0, SAMP_PER_W)], buf)

    @pl.loop(0, SAMP_PER_W)
    def _row(j):
        for c in _COLS:
            flat[pl.ds(j * HIST + c, 16)] = buf[j, pl.ds(c, 16)]

    pltpu.sync_copy(flat, x2_hbm.at[pl.ds(s0 * HIST, SAMP_PER_W * HIST)])


def _repack(x):
    mesh = plsc.VectorSubcoreMesh(
        core_axis_name="c", subcore_axis_name="s",
        num_cores=NUM_CORES, num_subcores=NUM_SUBCORES)
    return pl.kernel(
        _depad_body,
        out_type=jax.ShapeDtypeStruct((TOTAL,), jnp.int32),
        mesh=mesh,
        scratch_types=[
            pltpu.VMEM((SAMP_PER_W, HIST), jnp.int32),
            pltpu.VMEM((SAMP_PER_W * HIST,), jnp.int32),
            pltpu.SemaphoreType.DMA,
        ],
        compiler_params=pltpu.CompilerParams(use_tc_tiling_on_sc=True),
    )(x)


def _gather_body(x2_hbm, table_hbm, out_hbm,
                 idx_v, rows0, rows1, gsem0, gsem1, ssem0, ssem1):
    w = lax.axis_index("s") * NUM_CORES + lax.axis_index("c")
    base = w * B_PER_W

    pltpu.sync_copy(x2_hbm.at[pl.ds(base, B_PER_W)], idx_v)

    bufs = (rows0, rows1)
    gsems = (gsem0, gsem1)
    ssems = (ssem0, ssem1)

    def start_gather(s, b):
        for j in range(NGPER):
            pltpu.async_copy(
                table_hbm.at[idx_v.at[pl.ds((s * NGPER + j) * 128, 128)]],
                bufs[b].at[pl.ds(j * 128, 128)],
                gsems[b])

    def wait_gather(b):
        pltpu.make_async_copy(
            out_hbm.at[pl.ds(base, G)], bufs[b], gsems[b]).wait()

    def start_store(s, b):
        pltpu.async_copy(bufs[b], out_hbm.at[pl.ds(base + s * G, G)],
                         ssems[b])

    def wait_store(b):
        pltpu.make_async_copy(
            bufs[b], out_hbm.at[pl.ds(base, G)], ssems[b]).wait()

    start_gather(0, 0)
    wait_gather(0)
    start_gather(1, 1)
    start_store(0, 0)

    @pl.loop(0, NPAIR)
    def _pair(p):
        s = 1 + 2 * p
        for b in (1, 0):
            wait_gather(b)
            wait_store(1 - b)
            start_gather(s + 1, 1 - b)
            start_store(s, b)
            s = s + 1

    b_last = (NSTEP - 1) % 2
    wait_gather(b_last)
    wait_store(1 - b_last)
    start_store(NSTEP - 1, b_last)
    wait_store(b_last)


def _gather(x2, table):
    mesh = plsc.VectorSubcoreMesh(
        core_axis_name="c", subcore_axis_name="s",
        num_cores=NUM_CORES, num_subcores=NUM_SUBCORES)
    return pl.kernel(
        _gather_body,
        out_type=jax.ShapeDtypeStruct((TOTAL, EMBED_DIM), jnp.float32),
        mesh=mesh,
        scratch_types=[
            pltpu.VMEM((B_PER_W,), jnp.int32),
            pltpu.VMEM((G, EMBED_DIM), jnp.float32),
            pltpu.VMEM((G, EMBED_DIM), jnp.float32),
            pltpu.SemaphoreType.DMA,
            pltpu.SemaphoreType.DMA,
            pltpu.SemaphoreType.DMA,
            pltpu.SemaphoreType.DMA,
        ],
        compiler_params=pltpu.CompilerParams(use_tc_tiling_on_sc=False),
    )(x2, table)


@jax.jit
def _embedding_sc(x, table):
    table_sc = jex_layout.with_layout_constraint(
        table, jex_layout.Layout((0, 1), tiling=((16,),)))
    out2 = _gather(_repack(x), table_sc)
    out2 = jax.lax.optimization_barrier(out2)
    return out2.reshape(BATCH, HIST, EMBED_DIM)


def kernel(x, table):
    return _embedding_sc(x, table)
